# baseline (speedup 1.0000x reference)
import functools

import jax
import jax.numpy as jnp
from jax import lax
from jax.experimental import pallas as pl
from jax.experimental.pallas import tpu as pltpu
from jax.experimental.pallas import tpu_sc as plsc

EMB = 256
N_ROWS = 16384
BETA = 0.25

ROW_BLK = 512
K_BLK = 1024

SC_WORKERS = 32
SC_CHUNK = 256


def _dist_body(x_ref, d_ref, xn_ref, dn_ref, idx_ref, *, kdim):
    x = x_ref[...]
    xn = xn_ref[...]
    m = jnp.full((ROW_BLK, 1), jnp.inf, jnp.float32)
    a = jnp.zeros((ROW_BLK, 1), jnp.int32)
    kb = min(K_BLK, kdim)
    for c in range(0, kdim, kb):
        sim = lax.dot_general(
            x, d_ref[:, c:c + kb], (((1,), (0,)), ((), ())),
            preferred_element_type=jnp.float32)
        v = (xn + dn_ref[0:1, c:c + kb]) - 2.0 * sim
        cm = jnp.min(v, axis=1, keepdims=True)
        cols = lax.broadcasted_iota(jnp.int32, v.shape, 1) + c
        ca = jnp.min(jnp.where(v == cm, cols, jnp.int32(2 ** 30)),
                     axis=1, keepdims=True)
        upd = cm < m
        a = jnp.where(upd, ca, a)
        m = jnp.where(upd, cm, m)
    idx_ref[...] = a


def _dist_argmin(xf, d, xn, dn):
    kdim = d.shape[1]
    return pl.pallas_call(
        functools.partial(_dist_body, kdim=kdim),
        grid=(N_ROWS // ROW_BLK,),
        in_specs=[
            pl.BlockSpec((ROW_BLK, EMB), lambda i: (i, 0)),
            pl.BlockSpec((EMB, kdim), lambda i: (0, 0)),
            pl.BlockSpec((ROW_BLK, 1), lambda i: (i, 0)),
            pl.BlockSpec((1, kdim), lambda i: (0, 0)),
        ],
        out_specs=pl.BlockSpec((ROW_BLK, 1), lambda i: (i, 0)),
        out_shape=jax.ShapeDtypeStruct((N_ROWS, 1), jnp.int32),
    )(xf, d, xn, dn)


def _sc_gather(table, idx):
    b_per_w = N_ROWS // SC_WORKERS
    mesh = plsc.VectorSubcoreMesh(core_axis_name="c", subcore_axis_name="s")

    @functools.partial(
        pl.kernel, mesh=mesh,
        out_type=jax.ShapeDtypeStruct((N_ROWS, EMB), jnp.float32),
        scratch_types=[
            pltpu.VMEM((SC_CHUNK,), jnp.int32),
            pltpu.VMEM((SC_CHUNK, EMB), jnp.float32),
            pltpu.SemaphoreType.DMA,
        ],
    )
    def k(table_hbm, idx_hbm, out_hbm, idx_v, rows_v, sem):
        wid = lax.axis_index("s") * 2 + lax.axis_index("c")
        base = wid * b_per_w
        for c in range(0, b_per_w, SC_CHUNK):
            pltpu.sync_copy(idx_hbm.at[pl.ds(base + c, SC_CHUNK)], idx_v)
            pltpu.async_copy(table_hbm.at[idx_v], rows_v, sem).wait()
            pltpu.sync_copy(rows_v, out_hbm.at[pl.ds(base + c, SC_CHUNK)])

    return k(table, idx)


def _combine_body(gam_ref, al_ref, x_ref, g0, g1, g2, g3, g4,
                  out_ref, loss_ref):
    i = pl.program_id(0)
    al = al_ref[0]
    x = x_ref[...]
    acc = jnp.zeros(x.shape, jnp.float32)
    lsum = jnp.float32(0.0)
    for j, g_ref in enumerate((g0, g1, g2, g3, g4)):
        q = al * g_ref[...].astype(jnp.bfloat16).astype(jnp.float32)
        acc = acc + gam_ref[j] * q
        dif = x - q
        lsum = lsum + gam_ref[j] * jnp.sum(dif * dif)
    out_ref[...] = acc

    @pl.when(i == 0)
    def _():
        loss_ref[...] = jnp.zeros((1, 1), jnp.float32)

    loss_ref[...] += jnp.reshape(lsum * ((1.0 + BETA) / (N_ROWS * EMB)),
                                 (1, 1))


def _combine(xf, gs, vq_gamma, vq_alpha):
    blk = 1024
    grid = (N_ROWS // blk,)
    row_spec = pl.BlockSpec((blk, EMB), lambda i: (i, 0))
    out, loss = pl.pallas_call(
        _combine_body,
        grid=grid,
        in_specs=[
            pl.BlockSpec(memory_space=pltpu.SMEM),
            pl.BlockSpec(memory_space=pltpu.SMEM),
            row_spec, row_spec, row_spec, row_spec, row_spec, row_spec,
        ],
        out_specs=[
            pl.BlockSpec((blk, EMB), lambda i: (i, 0)),
            pl.BlockSpec((1, 1), lambda i: (0, 0)),
        ],
        out_shape=[
            jax.ShapeDtypeStruct((N_ROWS, EMB), jnp.float32),
            jax.ShapeDtypeStruct((1, 1), jnp.float32),
        ],
    )(vq_gamma, vq_alpha, xf, *gs)
    return out, loss


def kernel(x, dict0, dict1, dict2, dict3, dict4, vq_alpha, vq_gamma):
    dicts = [dict0, dict1, dict2, dict3, dict4]
    xf = x.reshape(-1, EMB)
    xn = jnp.sum(xf ** 2, axis=1, keepdims=True)
    xb = xf.astype(jnp.bfloat16)
    gs = []
    for d in dicts:
        dn = jnp.sum(d ** 2, axis=0, keepdims=True)
        db = d.astype(jnp.bfloat16)
        idx = _dist_argmin(xb, db, xn, dn)
        gs.append(_sc_gather(d.T, idx.reshape(N_ROWS)))
    out, loss = _combine(xf, gs, vq_gamma, vq_alpha.reshape(1))
    return out.reshape(x.shape), loss[0, 0]


# speedup vs baseline: 1.1154x; 1.1154x over previous
import functools

import jax
import jax.numpy as jnp
from jax import lax
from jax.experimental import pallas as pl
from jax.experimental.pallas import tpu as pltpu
from jax.experimental.pallas import tpu_sc as plsc

EMB = 256
N_ROWS = 16384
BETA = 0.25

ROW_BLK = 512
K_BLK = 1024

SC_WORKERS = 32
SC_CHUNK = 256


def _dist_body(x_ref, d_ref, xn_ref, dn_ref, idx_ref, *, kdim):
    x = x_ref[...]
    xn = xn_ref[...]
    kb = min(K_BLK, kdim)
    m_vec = jnp.full((ROW_BLK, kb), jnp.inf, jnp.float32)
    c_vec = jnp.zeros((ROW_BLK, kb), jnp.int32)
    for ci, c in enumerate(range(0, kdim, kb)):
        sim2 = lax.dot_general(
            x, d_ref[:, c:c + kb], (((1,), (0,)), ((), ())),
            preferred_element_type=jnp.float32)
        v = (xn + dn_ref[0:1, c:c + kb]) + sim2
        better = v < m_vec
        m_vec = jnp.where(better, v, m_vec)
        c_vec = jnp.where(better, jnp.int32(ci), c_vec)
    m = jnp.min(m_vec, axis=1, keepdims=True)
    kcand = c_vec * jnp.int32(kb) + lax.broadcasted_iota(
        jnp.int32, (ROW_BLK, kb), 1)
    a = jnp.min(jnp.where(m_vec == m, kcand, jnp.int32(2 ** 30)),
                axis=1, keepdims=True)
    idx_ref[...] = a


def _dist_argmin(xf, d, xn, dn):
    kdim = d.shape[1]
    return pl.pallas_call(
        functools.partial(_dist_body, kdim=kdim),
        grid=(N_ROWS // ROW_BLK,),
        in_specs=[
            pl.BlockSpec((ROW_BLK, EMB), lambda i: (i, 0)),
            pl.BlockSpec((EMB, kdim), lambda i: (0, 0)),
            pl.BlockSpec((ROW_BLK, 1), lambda i: (i, 0)),
            pl.BlockSpec((1, kdim), lambda i: (0, 0)),
        ],
        out_specs=pl.BlockSpec((ROW_BLK, 1), lambda i: (i, 0)),
        out_shape=jax.ShapeDtypeStruct((N_ROWS, 1), jnp.int32),
    )(xf, d, xn, dn)


def _sc_gather(table, idx):
    b_per_w = N_ROWS // SC_WORKERS
    mesh = plsc.VectorSubcoreMesh(core_axis_name="c", subcore_axis_name="s")

    @functools.partial(
        pl.kernel, mesh=mesh,
        out_type=jax.ShapeDtypeStruct((N_ROWS, EMB), jnp.float32),
        scratch_types=[
            pltpu.VMEM((SC_CHUNK,), jnp.int32),
            pltpu.VMEM((SC_CHUNK, EMB), jnp.float32),
            pltpu.SemaphoreType.DMA,
        ],
    )
    def k(table_hbm, idx_hbm, out_hbm, idx_v, rows_v, sem):
        wid = lax.axis_index("s") * 2 + lax.axis_index("c")
        base = wid * b_per_w
        for c in range(0, b_per_w, SC_CHUNK):
            pltpu.sync_copy(idx_hbm.at[pl.ds(base + c, SC_CHUNK)], idx_v)
            pltpu.async_copy(table_hbm.at[idx_v], rows_v, sem).wait()
            pltpu.sync_copy(rows_v, out_hbm.at[pl.ds(base + c, SC_CHUNK)])

    return k(table, idx)


def _combine_body(gam_ref, al_ref, x_ref, g0, g1, g2, g3, g4,
                  out_ref, loss_ref):
    i = pl.program_id(0)
    al = al_ref[0]
    x = x_ref[...]
    acc = jnp.zeros(x.shape, jnp.float32)
    lsum = jnp.float32(0.0)
    for j, g_ref in enumerate((g0, g1, g2, g3, g4)):
        q = al * g_ref[...].astype(jnp.bfloat16).astype(jnp.float32)
        acc = acc + gam_ref[j] * q
        dif = x - q
        lsum = lsum + gam_ref[j] * jnp.sum(dif * dif)
    out_ref[...] = acc

    @pl.when(i == 0)
    def _():
        loss_ref[...] = jnp.zeros((1, 1), jnp.float32)

    loss_ref[...] += jnp.reshape(lsum * ((1.0 + BETA) / (N_ROWS * EMB)),
                                 (1, 1))


def _combine(xf, gs, vq_gamma, vq_alpha):
    blk = 1024
    grid = (N_ROWS // blk,)
    row_spec = pl.BlockSpec((blk, EMB), lambda i: (i, 0))
    out, loss = pl.pallas_call(
        _combine_body,
        grid=grid,
        in_specs=[
            pl.BlockSpec(memory_space=pltpu.SMEM),
            pl.BlockSpec(memory_space=pltpu.SMEM),
            row_spec, row_spec, row_spec, row_spec, row_spec, row_spec,
        ],
        out_specs=[
            pl.BlockSpec((blk, EMB), lambda i: (i, 0)),
            pl.BlockSpec((1, 1), lambda i: (0, 0)),
        ],
        out_shape=[
            jax.ShapeDtypeStruct((N_ROWS, EMB), jnp.float32),
            jax.ShapeDtypeStruct((1, 1), jnp.float32),
        ],
    )(vq_gamma, vq_alpha, xf, *gs)
    return out, loss


def kernel(x, dict0, dict1, dict2, dict3, dict4, vq_alpha, vq_gamma):
    dicts = [dict0, dict1, dict2, dict3, dict4]
    xf = x.reshape(-1, EMB)
    xn = jnp.sum(xf ** 2, axis=1, keepdims=True)
    xb = (xf * (-2.0)).astype(jnp.bfloat16)
    gs = []
    for d in dicts:
        dn = jnp.sum(d ** 2, axis=0, keepdims=True)
        db = d.astype(jnp.bfloat16)
        idx = _dist_argmin(xb, db, xn, dn)
        gs.append(_sc_gather(d.T, idx.reshape(N_ROWS)))
    out, loss = _combine(xf, gs, vq_gamma, vq_alpha.reshape(1))
    return out.reshape(x.shape), loss[0, 0]
